# Initial kernel scaffold; baseline (speedup 1.0000x reference)
import jax
import jax.numpy as jnp
from jax.experimental import pallas as pl


def kernel(features, labels):
    raise NotImplementedError("write your pallas kernel here")


# speedup vs baseline: 3.4844x; 3.4844x over previous
import numpy as np

import jax
import jax.numpy as jnp
from jax import lax
from jax.experimental import pallas as pl
from jax.experimental.pallas import tpu as pltpu
from jax.experimental.pallas import tpu_sc as plsc

N_ROWS = 16384
D = 64
C = 64
MARGIN_ = 2.0

NUM_CORES = 2
NUM_SUBCORES = 16
NW = NUM_CORES * NUM_SUBCORES
RPW = N_ROWS // NW
L = 16
UNROLL = 4

_RSQRT_MAGIC = np.int32(0x5F3759DF)

_GDN = lax.GatherDimensionNumbers(
    offset_dims=(), collapsed_slice_dims=(0,), start_index_map=(0,))


def _permute(v, idx):
    return lax.gather(
        v, idx[:, None], dimension_numbers=_GDN, slice_sizes=(1,),
        mode=lax.GatherScatterMode.PROMISE_IN_BOUNDS)


def _sc_body(features, labels, sums_out, cnt_out, sq_out, fvm, lvm, acc, cnt2, sqv):
    cid = lax.axis_index("c")
    sid = lax.axis_index("s")
    wid = sid * NUM_CORES + cid
    base = wid * RPW

    pltpu.sync_copy(features.at[pl.ds(base, RPW)], fvm)
    pltpu.sync_copy(labels.at[pl.ds(base, RPW)], lvm)

    zeros = jnp.zeros((L,), jnp.float32)
    ones = jnp.ones((L,), jnp.float32)
    iota_l = lax.iota(jnp.int32, L)

    def zero_body(k, carry):
        for j in range(D // L):
            acc[k, pl.ds(L * j, L)] = zeros
        cnt2[k, pl.ds(0, L)] = zeros
        return carry

    lax.fori_loop(0, C, zero_body, 0)

    def row_body(i, sqfn):
        lab_vec = lvm[pl.ds(i * L, L)]
        for u in range(L):
            r = i * L + u
            lab = lab_vec[u]
            f = [fvm[r, pl.ds(L * j, L)] for j in range(D // L)]
            t = f[0] * f[0]
            for j in range(1, D // L):
                t = t + f[j] * f[j]
            s = t
            for k in (8, 4, 2, 1):
                s = s + _permute(s, iota_l ^ k)
            s = jnp.maximum(s, 1e-24)
            bits = plsc.bitcast(s, jnp.int32)
            y = plsc.bitcast(_RSQRT_MAGIC - (bits >> 1), jnp.float32)
            for _ in range(3):
                y = y * (1.5 - 0.5 * s * y * y)
            sqfn = sqfn + t * (y * y)
            for j in range(D // L):
                plsc.addupdate(acc.at[lab, pl.ds(L * j, L)], f[j] * y)
            plsc.addupdate(cnt2.at[lab], ones)
        return sqfn

    sqfn = lax.fori_loop(0, RPW // L, row_body, zeros)
    sqv[pl.ds(0, L)] = sqfn

    pltpu.sync_copy(acc, sums_out.at[wid])
    pltpu.sync_copy(cnt2, cnt_out.at[wid])
    pltpu.sync_copy(sqv, sq_out.at[wid])


_sc_segment = pl.kernel(
    _sc_body,
    out_type=[
        jax.ShapeDtypeStruct((NW, C, D), jnp.float32),
        jax.ShapeDtypeStruct((NW, C, L), jnp.float32),
        jax.ShapeDtypeStruct((NW, L), jnp.float32),
    ],
    mesh=plsc.VectorSubcoreMesh(
        core_axis_name="c", subcore_axis_name="s",
        num_cores=NUM_CORES, num_subcores=NUM_SUBCORES,
    ),
    scratch_types=[
        pltpu.VMEM((RPW, D), jnp.float32),
        pltpu.VMEM((RPW,), jnp.int32),
        pltpu.VMEM((C, D), jnp.float32),
        pltpu.VMEM((C, L), jnp.float32),
        pltpu.VMEM((L,), jnp.float32),
    ],
    compiler_params=pltpu.CompilerParams(needs_layout_passes=False),
)


def _epi_body(sums_ref, cnt_ref, sq_ref, out_ref):
    sums = jnp.sum(sums_ref[...], axis=0)
    counts = jnp.sum(cnt_ref[...], axis=0)[:, 0]
    sqtot = jnp.sum(sq_ref[...])
    csafe = jnp.maximum(counts, 1.0)
    centers = sums / csafe[:, None]
    cnorm2 = jnp.sum(centers * centers, axis=1)
    intra = (sqtot - jnp.sum(counts * cnorm2)) / jnp.float32(N_ROWS)
    gram = jnp.dot(centers, centers.T, preferred_element_type=jnp.float32)
    d2 = cnorm2[:, None] + cnorm2[None, :] - 2.0 * gram
    d2 = jnp.maximum(d2, 0.0)
    row = lax.broadcasted_iota(jnp.int32, (C, C), 0)
    col = lax.broadcasted_iota(jnp.int32, (C, C), 1)
    pres = counts > 0.5
    mask = (row != col) & pres[:, None] & pres[None, :]
    min_d2 = jnp.min(jnp.where(mask, d2, jnp.float32(1e30)))
    min_inter = jnp.sqrt(min_d2)
    inter = jnp.maximum(MARGIN_ - min_inter, 0.0)
    sr = jnp.clip(min_inter / MARGIN_, 0.0, 1.0)
    loss = (1.0 + 2.0 * (1.0 - sr)) * intra + (2.0 * sr) * inter
    npres = jnp.sum(pres.astype(jnp.float32))
    loss = jnp.where(npres < 1.5, jnp.float32(0.0), loss)
    out_ref[...] = jnp.broadcast_to(loss, (1, 1))


_epilogue = pl.pallas_call(
    _epi_body,
    out_shape=jax.ShapeDtypeStruct((1, 1), jnp.float32),
)


@jax.jit
def kernel(features, labels):
    sums, cnt, sq = _sc_segment(features, labels)
    return _epilogue(sums, cnt, sq)[0, 0]
